# baseline (speedup 1.0000x reference)
import functools
import math

import jax
import jax.numpy as jnp
from jax import lax
from jax.experimental import pallas as pl
from jax.experimental.pallas import tpu as pltpu
from jax.experimental.pallas import tpu_sc as plsc

EMB = 32
SCALE = math.sqrt(EMB)

NC = 2
NS = 16
NW = NC * NS

G = 128
L = 16


def _make_emb_kernel(S, B0):
    W = B0 // NW
    CPR = W // G
    nchunks = S * CPR
    mesh = plsc.VectorSubcoreMesh(core_axis_name="c", subcore_axis_name="s")

    @functools.partial(
        pl.kernel,
        mesh=mesh,
        out_type=jax.ShapeDtypeStruct((S, EMB, B0), jnp.float32),
        scratch_types=[
            pltpu.VMEM((S, W), jnp.int32),
            pltpu.VMEM((2, G, EMB), jnp.float32),
            pltpu.VMEM((2, 1, EMB, G), jnp.float32),
        ]
        + [pltpu.SemaphoreType.DMA] * 5,
        compiler_params=pltpu.CompilerParams(
            use_tc_tiling_on_sc=False, needs_layout_passes=False),
    )
    def emb_kernel(tokT_hbm, table_hbm, out_hbm, idx_v, rows_v, tr_v,
                   sem_i, sem_g0, sem_g1, sem_s0, sem_s1):
        sem_g = (sem_g0, sem_g1)
        sem_s = (sem_s0, sem_s1)
        wid = lax.axis_index("s") * NC + lax.axis_index("c")
        col0 = wid * W

        def ci_split(ci):
            return ci // CPR, ci % CPR

        def fire_gather(ci, b):
            i1, c = ci_split(ci)
            pltpu.async_copy(
                table_hbm.at[idx_v.at[i1, pl.ds(c * G, G)]],
                rows_v.at[b], sem_g[b])

        def wait_gather(b):
            pltpu.make_async_copy(
                table_hbm.at[idx_v.at[0, pl.ds(0, G)]],
                rows_v.at[b], sem_g[b]).wait()

        def fire_store(ci, b):
            i1, c = ci_split(ci)
            pltpu.async_copy(
                tr_v.at[b],
                out_hbm.at[pl.ds(i1, 1), pl.ds(0, EMB),
                           pl.ds(col0 + c * G, G)],
                sem_s[b])

        def wait_store(b):
            pltpu.make_async_copy(
                tr_v.at[b],
                out_hbm.at[pl.ds(0, 1), pl.ds(0, EMB), pl.ds(col0, G)],
                sem_s[b]).wait()

        lane = lax.iota(jnp.int32, L)
        row_idx = [lane + (l * L) for l in range(G // L)]

        def transpose_scale(b):
            for e in range(EMB):
                col = jnp.full((L,), e, jnp.int32)
                for l in range(G // L):
                    v = plsc.load_gather(rows_v.at[b], [row_idx[l], col])
                    tr_v[b, 0, e, pl.ds(l * L, L)] = v * SCALE

        pltpu.async_copy(
            tokT_hbm.at[pl.ds(0, S), pl.ds(col0, W)], idx_v, sem_i)
        pltpu.make_async_copy(
            tokT_hbm.at[pl.ds(0, S), pl.ds(col0, W)], idx_v, sem_i).wait()
        fire_gather(0, 0)
        fire_gather(1, 1)
        fire_store(0, 0)
        fire_store(1, 1)

        def half(ci, b):
            wait_store(b)
            wait_gather(b)
            transpose_scale(b)
            fire_gather(jnp.minimum(ci + 2, nchunks - 1), b)
            fire_store(ci, b)

        def pair(m, carry):
            half(2 * m, 0)
            half(2 * m + 1, 1)
            return carry

        lax.fori_loop(0, nchunks // 2, pair, 0)
        wait_store(0)
        wait_store(1)
        wait_gather(0)
        wait_gather(1)

    return emb_kernel


def kernel(tokens, table):
    B0, S = tokens.shape
    assert B0 % (NW * G) == 0
    tokT = tokens.T.astype(jnp.int32)
    out_t = _make_emb_kernel(S, B0)(tokT, table)
    return jnp.transpose(out_t, (2, 0, 1))


# speedup vs baseline: 1.6242x; 1.6242x over previous
import functools
import math

import jax
import jax.numpy as jnp
from jax import lax
from jax.experimental import pallas as pl
from jax.experimental.pallas import tpu as pltpu
from jax.experimental.pallas import tpu_sc as plsc

EMB = 32
SCALE = math.sqrt(EMB)

NC = 2
NS = 16
NW = NC * NS

G = 128
K = 8
C = K * G
NSLOT = 3
U = 8


def _make_emb_kernel(B, b_per_w, nsteps):
    mesh = plsc.VectorSubcoreMesh(core_axis_name="c", subcore_axis_name="s")

    @functools.partial(
        pl.kernel,
        mesh=mesh,
        out_type=jax.ShapeDtypeStruct((B, EMB), jnp.float32),
        scratch_types=[
            pltpu.VMEM((NSLOT, K, G), jnp.int32),
            pltpu.VMEM((NSLOT, C, EMB), jnp.float32),
        ]
        + [pltpu.SemaphoreType.DMA] * (3 * NSLOT),
        compiler_params=pltpu.CompilerParams(use_tc_tiling_on_sc=False),
    )
    def emb_kernel(tok_hbm, table_hbm, out_hbm, idx_v, rows_v, *sems):
        sem_g = sems[0:NSLOT]
        sem_s = sems[NSLOT:2 * NSLOT]
        sem_i = sems[2 * NSLOT:3 * NSLOT]
        wid = lax.axis_index("s") * NC + lax.axis_index("c")
        w_base = wid * b_per_w

        def tok_rows(s):
            return pl.multiple_of((w_base + s * C) // G, 8)

        def fire_idx(s, b):
            return pltpu.async_copy(
                tok_hbm.at[pl.ds(tok_rows(s), K)], idx_v.at[b], sem_i[b])

        def fire_gathers(s, b):
            for j in range(K):
                pltpu.async_copy(
                    table_hbm.at[idx_v.at[b, j]],
                    rows_v.at[b, pl.ds(j * G, G)],
                    sem_g[b],
                )

        def wait_gathers(b):
            for j in range(K):
                pltpu.make_async_copy(
                    table_hbm.at[idx_v.at[b, j]],
                    rows_v.at[b, pl.ds(j * G, G)],
                    sem_g[b],
                ).wait()

        def fire_store(s, b):
            return pltpu.async_copy(
                rows_v.at[b], out_hbm.at[pl.ds(w_base + s * C, C)], sem_s[b])

        def wait_store(b):
            pltpu.make_async_copy(
                rows_v.at[b], out_hbm.at[pl.ds(w_base, C)], sem_s[b]).wait()

        def wait_idx(b):
            pltpu.make_async_copy(
                tok_hbm.at[pl.ds(tok_rows(0), K)], idx_v.at[b], sem_i[b]).wait()

        def scale(b):
            def body(i, carry):
                r0 = i * U
                for r in range(U):
                    rows_v[b, r0 + r, pl.ds(0, 16)] = (
                        rows_v[b, r0 + r, pl.ds(0, 16)] * SCALE)
                    rows_v[b, r0 + r, pl.ds(16, 16)] = (
                        rows_v[b, r0 + r, pl.ds(16, 16)] * SCALE)
                return carry

            lax.fori_loop(0, C // U, body, 0)

        fire_idx(0, 0)
        fire_idx(1, 1)
        fire_idx(2, 2)
        wait_idx(0)
        fire_gathers(0, 0)
        fire_store(1, 1)
        fire_store(2, 2)

        def half(s, b, b1):
            wait_idx(b1)
            wait_store(b1)
            fire_gathers(s + 1, b1)
            wait_gathers(b)
            scale(b)
            fire_store(s, b)
            fire_idx(jnp.minimum(s + 3, nsteps - 1), b)

        def triple(t, carry):
            s = 3 * t
            half(s, 0, 1)
            half(s + 1, 1, 2)
            half(s + 2, 2, 0)
            return carry

        lax.fori_loop(0, (nsteps - 1) // 3, triple, 0)
        s_last = nsteps - 1
        wait_gathers(0)
        scale(0)
        fire_store(s_last, 0)
        wait_store(1)
        wait_store(2)
        wait_store(0)
        wait_idx(1)
        wait_idx(2)

    return emb_kernel


def kernel(tokens, table):
    B0, S = tokens.shape
    B = B0 * S
    assert B % (NW * C) == 0
    b_per_w = B // NW
    nsteps = b_per_w // C
    assert nsteps % 3 == 1
    tok2d = tokens.reshape(B // G, G).astype(jnp.int32)
    out = _make_emb_kernel(B, b_per_w, nsteps)(tok2d, table)
    return out.reshape(B0, S, EMB)
